# baseline (speedup 1.0000x reference)
import jax
import jax.numpy as jnp
from jax.experimental import pallas as pl
from jax.experimental.pallas import tpu as pltpu

CH = 512
NBUF = 4


def kernel(code_embeddings, language_embed, lang_idx):
    B, S, D = code_embeddings.shape
    NK = S // CH
    assert NK * CH == S

    def body(idx_ref, table_ref, code_hbm, out_hbm,
             in_bufs, out_bufs, slab0, in_sems, out_sems, slab0_sem):
        lang = table_ref[idx_ref[0]]
        slab0[...] = jnp.broadcast_to(lang[None, None], (1, B, D))
        slab0_cp = pltpu.make_async_copy(slab0, out_hbm.at[pl.ds(0, 1)], slab0_sem)
        slab0_cp.start()

        def in_copy(k, slot, b):
            return pltpu.make_async_copy(
                code_hbm.at[b, pl.ds(k * CH, CH)], in_bufs.at[slot, b], in_sems.at[slot])

        def out_copy(k, slot):
            return pltpu.make_async_copy(
                out_bufs.at[slot], out_hbm.at[pl.ds(1 + k * CH, CH)], out_sems.at[slot])

        for k in range(min(NBUF, NK)):
            for b in range(B):
                in_copy(k, k % NBUF, b).start()

        for k in range(NK):
            slot = k % NBUF
            for b in range(B):
                in_copy(k, slot, b).wait()
            if k >= NBUF:
                out_copy(k - NBUF, slot).wait()
            x = in_bufs[slot]
            out_bufs[slot] = x.transpose(1, 0, 2)
            out_copy(k, slot).start()
            if k + NBUF < NK:
                for b in range(B):
                    in_copy(k + NBUF, slot, b).start()

        for k in range(max(NK - NBUF, 0), NK):
            out_copy(k, k % NBUF).wait()
        slab0_cp.wait()

    out_sm = pl.pallas_call(
        body,
        in_specs=[
            pl.BlockSpec(memory_space=pltpu.SMEM),
            pl.BlockSpec(memory_space=pltpu.VMEM),
            pl.BlockSpec(memory_space=pl.ANY),
        ],
        out_specs=pl.BlockSpec(memory_space=pl.ANY),
        out_shape=jax.ShapeDtypeStruct((S + 1, B, D), code_embeddings.dtype),
        compiler_params=pltpu.CompilerParams(vmem_limit_bytes=60 * 1024 * 1024),
        scratch_shapes=[
            pltpu.VMEM((NBUF, B, CH, D), code_embeddings.dtype),
            pltpu.VMEM((NBUF, CH, B, D), code_embeddings.dtype),
            pltpu.VMEM((1, B, D), code_embeddings.dtype),
            pltpu.SemaphoreType.DMA((NBUF,)),
            pltpu.SemaphoreType.DMA((NBUF,)),
            pltpu.SemaphoreType.DMA,
        ],
    )(lang_idx, language_embed, code_embeddings)

    return jnp.swapaxes(out_sm, 0, 1)


# speedup vs baseline: 1.0192x; 1.0192x over previous
import jax
import jax.numpy as jnp
from jax.experimental import pallas as pl
from jax.experimental.pallas import tpu as pltpu

CH = 1024
NBUF = 2


def kernel(code_embeddings, language_embed, lang_idx):
    B, S, D = code_embeddings.shape
    NK = S // CH
    assert NK * CH == S

    def body(idx_ref, table_ref, code_hbm, out_hbm,
             in_bufs, out_bufs, slab0, in_sems, out_sems, slab0_sem):
        lang = table_ref[idx_ref[0]]
        slab0[...] = jnp.broadcast_to(lang[None, None], (1, B, D))
        slab0_cp = pltpu.make_async_copy(slab0, out_hbm.at[pl.ds(0, 1)], slab0_sem)
        slab0_cp.start()

        def in_copy(k, slot, b):
            return pltpu.make_async_copy(
                code_hbm.at[b, pl.ds(k * CH, CH)], in_bufs.at[slot, b], in_sems.at[slot])

        def out_copy(k, slot):
            return pltpu.make_async_copy(
                out_bufs.at[slot], out_hbm.at[pl.ds(1 + k * CH, CH)], out_sems.at[slot])

        for k in range(min(NBUF, NK)):
            for b in range(B):
                in_copy(k, k % NBUF, b).start()

        for k in range(NK):
            slot = k % NBUF
            for b in range(B):
                in_copy(k, slot, b).wait()
            if k >= NBUF:
                out_copy(k - NBUF, slot).wait()
            x = in_bufs[slot]
            out_bufs[slot] = x.transpose(1, 0, 2)
            out_copy(k, slot).start()
            if k + NBUF < NK:
                for b in range(B):
                    in_copy(k + NBUF, slot, b).start()

        for k in range(max(NK - NBUF, 0), NK):
            out_copy(k, k % NBUF).wait()
        slab0_cp.wait()

    out_sm = pl.pallas_call(
        body,
        in_specs=[
            pl.BlockSpec(memory_space=pltpu.SMEM),
            pl.BlockSpec(memory_space=pltpu.VMEM),
            pl.BlockSpec(memory_space=pl.ANY),
        ],
        out_specs=pl.BlockSpec(memory_space=pl.ANY),
        out_shape=jax.ShapeDtypeStruct((S + 1, B, D), code_embeddings.dtype),
        compiler_params=pltpu.CompilerParams(vmem_limit_bytes=60 * 1024 * 1024),
        scratch_shapes=[
            pltpu.VMEM((NBUF, B, CH, D), code_embeddings.dtype),
            pltpu.VMEM((NBUF, CH, B, D), code_embeddings.dtype),
            pltpu.VMEM((1, B, D), code_embeddings.dtype),
            pltpu.SemaphoreType.DMA((NBUF,)),
            pltpu.SemaphoreType.DMA((NBUF,)),
            pltpu.SemaphoreType.DMA,
        ],
    )(lang_idx, language_embed, code_embeddings)

    return jnp.swapaxes(out_sm, 0, 1)


# speedup vs baseline: 1.0241x; 1.0048x over previous
import jax
import jax.numpy as jnp
from jax.experimental import pallas as pl
from jax.experimental.pallas import tpu as pltpu

NBUF = 2
CHMAX = 1024


def _chunk_sizes(S):
    sizes = [256]
    while S - sum(sizes) > CHMAX + 512:
        sizes.append(CHMAX)
    rem = S - sum(sizes)
    sizes += [rem - 256, 256]
    assert sum(sizes) == S and all(0 < s <= CHMAX and s % 8 == 0 for s in sizes)
    return sizes


def kernel(code_embeddings, language_embed, lang_idx):
    B, S, D = code_embeddings.shape
    sizes = _chunk_sizes(S)
    starts = [sum(sizes[:i]) for i in range(len(sizes))]
    N = len(sizes)

    def body(idx_ref, table_ref, code_hbm, out_hbm,
             in_bufs, out_bufs, slab0, in_sems, out_sems, slab0_sem):
        lang = table_ref[idx_ref[0]]
        slab0[...] = jnp.broadcast_to(lang[None, None], (1, B, D))
        slab0_cp = pltpu.make_async_copy(slab0, out_hbm.at[pl.ds(0, 1)], slab0_sem)
        slab0_cp.start()

        def in_copy(i, slot, b):
            return pltpu.make_async_copy(
                code_hbm.at[b, pl.ds(starts[i], sizes[i])],
                in_bufs.at[slot, b, pl.ds(0, sizes[i])],
                in_sems.at[slot])

        def out_copy(i, slot):
            return pltpu.make_async_copy(
                out_bufs.at[slot, pl.ds(0, sizes[i])],
                out_hbm.at[pl.ds(1 + starts[i], sizes[i])],
                out_sems.at[slot])

        for i in range(min(NBUF, N)):
            for b in range(B):
                in_copy(i, i % NBUF, b).start()

        for i in range(N):
            slot = i % NBUF
            for b in range(B):
                in_copy(i, slot, b).wait()
            if i >= NBUF:
                out_copy(i - NBUF, slot).wait()
            x = in_bufs[slot, :, pl.ds(0, sizes[i])]
            out_bufs[slot, pl.ds(0, sizes[i])] = x.transpose(1, 0, 2)
            out_copy(i, slot).start()
            if i + NBUF < N:
                for b in range(B):
                    in_copy(i + NBUF, slot, b).start()

        for i in range(max(N - NBUF, 0), N):
            out_copy(i, i % NBUF).wait()
        slab0_cp.wait()

    out_sm = pl.pallas_call(
        body,
        in_specs=[
            pl.BlockSpec(memory_space=pltpu.SMEM),
            pl.BlockSpec(memory_space=pltpu.VMEM),
            pl.BlockSpec(memory_space=pl.ANY),
        ],
        out_specs=pl.BlockSpec(memory_space=pl.ANY),
        out_shape=jax.ShapeDtypeStruct((S + 1, B, D), code_embeddings.dtype),
        compiler_params=pltpu.CompilerParams(vmem_limit_bytes=60 * 1024 * 1024),
        scratch_shapes=[
            pltpu.VMEM((NBUF, B, CHMAX, D), code_embeddings.dtype),
            pltpu.VMEM((NBUF, CHMAX, B, D), code_embeddings.dtype),
            pltpu.VMEM((1, B, D), code_embeddings.dtype),
            pltpu.SemaphoreType.DMA((NBUF,)),
            pltpu.SemaphoreType.DMA((NBUF,)),
            pltpu.SemaphoreType.DMA,
        ],
    )(lang_idx, language_embed, code_embeddings)

    return jnp.swapaxes(out_sm, 0, 1)
